# baseline (speedup 1.0000x reference)
import functools

import jax
import jax.numpy as jnp
from jax import lax
from jax.experimental import pallas as pl
from jax.experimental.pallas import tpu as pltpu
from jax.experimental.pallas import tpu_sc as plsc

_INFO = plsc.get_sparse_core_info()
_NC = _INFO.num_cores
_NS = _INFO.num_subcores
_NW = _NC * _NS
_LN2 = 0.6931471805599453
_BI = 128
_BJ = 8


@functools.cache
def _make_gather(nb, nl, v, chunk):
    blk = _BI * _BJ
    nit = nb // _BI
    n = nb * nl
    n_blocks = n // blk
    per_w = n_blocks // _NW
    bpc = chunk // blk
    n_chunks = per_w // bpc
    assert chunk % blk == 0 and per_w % bpc == 0 and n_chunks >= 2
    n_full = v // chunk
    tail_off = n_full * chunk
    tail = v - tail_off
    assert tail % 16 == 0
    plan = [[] for _ in range(_NS)]
    for c in range(n_full):
        plan[c % _NS].append((c * chunk, chunk))
    if tail:
        plan[_NS - 1].append((tail_off, tail))
    mesh = plsc.VectorSubcoreMesh(core_axis_name="c", subcore_axis_name="s")

    @functools.partial(
        pl.kernel,
        mesh=mesh,
        out_type=jax.ShapeDtypeStruct((n,), jnp.float32),
        scratch_types=[
            pltpu.VMEM_SHARED((v,), jnp.float32),
            pltpu.HBM((_NC * v,), jnp.float32),
            pltpu.VMEM((chunk,), jnp.int32),
            pltpu.VMEM((chunk,), jnp.int32),
            pltpu.VMEM((chunk,), jnp.int32),
            pltpu.VMEM((chunk,), jnp.float32),
            pltpu.VMEM((chunk,), jnp.float32),
            pltpu.VMEM((chunk,), jnp.float32),
            pltpu.SemaphoreType.DMA,
            pltpu.SemaphoreType.DMA,
            pltpu.SemaphoreType.DMA,
            pltpu.SemaphoreType.DMA,
            pltpu.SemaphoreType.DMA,
            pltpu.SemaphoreType.DMA,
            pltpu.SemaphoreType.DMA,
            pltpu.SemaphoreType.DMA,
            pltpu.SemaphoreType.DMA,
            pltpu.SemaphoreType.DMA,
            pltpu.SemaphoreType.DMA,
        ],
    )
    def gather_kernel(table_hbm, idx_hbm, out_hbm,
                      shared, tab_scr,
                      idx0, idx1, idx2, rows0, rows1, rows2,
                      si0, si1, si2, sg0, sg1, sg2, so0, so1, so2,
                      sh0, sh1):
        tid = lax.axis_index("s")
        cid = lax.axis_index("c")
        wid = tid * _NC + cid
        base_blk = wid * per_w
        base = base_blk * blk
        nbuf_n = 3
        idx_v = [idx0, idx1, idx2]
        rows_v = [rows0, rows1, rows2]
        si = [si0, si1, si2]
        sg = [sg0, sg1, sg2]
        so = [so0, so1, so2]
        fetches = [None] * nbuf_n
        gathers = [None] * nbuf_n
        stores = [None] * nbuf_n

        for i in range(nbuf_n):
            fetches[i] = pltpu.async_copy(
                idx_hbm.at[pl.ds(base + i * chunk, chunk)], idx_v[i], si[i])

        def compute(buf, off, size):
            def body(k, carry):
                s = pl.ds(k * 16, 16)
                g = lax.iota(jnp.int32, 16) + (off + k * 16)
                buf[s] = jnp.where(g == 0, 1.0, jnp.exp(buf[s] * _LN2))
                return carry

            lax.fori_loop(0, size // 16, body, 0)

        sh = [sh0, sh1]
        for t in range(_NS):
            @pl.when(tid == t)
            def _(t=t):
                seq = plan[t]
                ins = [None, None]
                outs = [None, None]
                outh = [None, None]
                ins[0] = pltpu.async_copy(
                    table_hbm.at[0, pl.ds(seq[0][0], seq[0][1])],
                    rows_v[0].at[pl.ds(0, seq[0][1])], sg[0])
                for ci, (off, size) in enumerate(seq):
                    b, nbuf = ci % 2, (ci + 1) % 2
                    if ci + 1 < len(seq):
                        noff, nsize = seq[ci + 1]
                        if outs[nbuf] is not None:
                            outs[nbuf].wait()
                            outh[nbuf].wait()
                        ins[nbuf] = pltpu.async_copy(
                            table_hbm.at[0, pl.ds(noff, nsize)],
                            rows_v[nbuf].at[pl.ds(0, nsize)], sg[nbuf])
                    ins[b].wait()
                    compute(rows_v[b], off, size)
                    outs[b] = pltpu.async_copy(
                        rows_v[b].at[pl.ds(0, size)],
                        shared.at[pl.ds(off, size)], so[b])
                    outh[b] = pltpu.async_copy(
                        rows_v[b].at[pl.ds(0, size)],
                        tab_scr.at[pl.ds(cid * v + off, size)], sh[b])
                for o in outs + outh:
                    if o is not None:
                        o.wait()

        plsc.subcore_barrier()

        def store_chunk(i, b):
            def body(k, carry):
                g = base_blk + i * bpc + k
                jt = g // nit
                it = g - jt * nit
                obase = jt * (_BJ * nb) + it * _BI
                for jj in range(_BJ):
                    pltpu.async_copy(
                        rows_v[b].at[pl.ds(k * blk + jj * _BI, _BI)],
                        out_hbm.at[pl.ds(obase + jj * nb, _BI)], so[b])
                return carry

            lax.fori_loop(0, bpc, body, 0)

        def wait_store(b):
            pltpu.make_async_copy(
                out_hbm.at[pl.ds(0, chunk)], rows_v[b], so[b]).wait()

        tabh = tab_scr.at[pl.ds(cid * v, v)]
        hbm_src = {1, 4, 7}

        def launch(j):
            src = tabh if j in hbm_src else shared
            jb = j % nbuf_n
            return pltpu.async_copy(src.at[idx_v[jb]], rows_v[jb], sg[jb])

        for j in range(nbuf_n - 1):
            fetches[j].wait()
            gathers[j] = launch(j)

        for i in range(n_chunks):
            b = i % nbuf_n
            g = i + nbuf_n - 1
            if g < n_chunks:
                gb = g % nbuf_n
                if stores[gb] is not None:
                    wait_store(gb)
                fetches[gb].wait()
                gathers[gb] = launch(g)
            gathers[b].wait()
            store_chunk(i, b)
            stores[b] = True
            if i + nbuf_n < n_chunks:
                fetches[b] = pltpu.async_copy(
                    idx_hbm.at[pl.ds(base + (i + nbuf_n) * chunk, chunk)],
                    idx_v[b], si[b])
        for j in range(nbuf_n):
            if stores[j] is not None:
                wait_store(j)

    return gather_kernel


def kernel(x, table):
    b, l = x.shape
    n = b * l
    v = table.shape[0]
    xb = x.reshape(b // _BI, _BI, l // _BJ, _BJ)
    xb = xb.transpose(2, 0, 3, 1).reshape(-1)
    out = _make_gather(b, l, v, 10240)(table.T, xb)
    return out.reshape(l, b, 1).transpose(1, 0, 2)


# speedup vs baseline: 1.6930x; 1.6930x over previous
import functools

import jax
import jax.numpy as jnp
from jax import lax
from jax.experimental import pallas as pl
from jax.experimental.pallas import tpu as pltpu
from jax.experimental.pallas import tpu_sc as plsc

_INFO = plsc.get_sparse_core_info()
_NC = _INFO.num_cores
_NS = _INFO.num_subcores
_NW = _NC * _NS
_LN2 = 0.6931471805599453
_BI = 128
_BJ = 8


@functools.cache
def _make_gather(nb, nl, v, chunk):
    blk = _BI * _BJ
    nit = nb // _BI
    n = nb * nl
    n_blocks = n // blk
    per_w = n_blocks // _NW
    bpc = chunk // blk
    n_chunks = per_w // bpc
    assert chunk % blk == 0 and per_w % bpc == 0 and n_chunks >= 2
    n_full = v // chunk
    tail_off = n_full * chunk
    tail = v - tail_off
    assert tail % 16 == 0
    plan = [[] for _ in range(_NS)]
    for c in range(n_full):
        plan[c % _NS].append((c * chunk, chunk))
    if tail:
        plan[_NS - 1].append((tail_off, tail))
    mesh = plsc.VectorSubcoreMesh(core_axis_name="c", subcore_axis_name="s")

    @functools.partial(
        pl.kernel,
        mesh=mesh,
        out_type=jax.ShapeDtypeStruct((n,), jnp.float32),
        scratch_types=[
            pltpu.VMEM_SHARED((v,), jnp.float32),
            pltpu.VMEM((chunk,), jnp.int32),
            pltpu.VMEM((chunk,), jnp.int32),
            pltpu.VMEM((chunk,), jnp.int32),
            pltpu.VMEM((chunk,), jnp.float32),
            pltpu.VMEM((chunk,), jnp.float32),
            pltpu.VMEM((chunk,), jnp.float32),
            pltpu.SemaphoreType.DMA,
            pltpu.SemaphoreType.DMA,
            pltpu.SemaphoreType.DMA,
            pltpu.SemaphoreType.DMA,
            pltpu.SemaphoreType.DMA,
            pltpu.SemaphoreType.DMA,
            pltpu.SemaphoreType.DMA,
            pltpu.SemaphoreType.DMA,
            pltpu.SemaphoreType.DMA,
        ],
    )
    def gather_kernel(table_hbm, idx_hbm, out_hbm,
                      shared,
                      idx0, idx1, idx2, rows0, rows1, rows2,
                      si0, si1, si2, sg0, sg1, sg2, so0, so1, so2):
        tid = lax.axis_index("s")
        cid = lax.axis_index("c")
        wid = tid * _NC + cid
        base_blk = wid * per_w
        base = base_blk * blk
        nbuf_n = 3
        idx_v = [idx0, idx1, idx2]
        rows_v = [rows0, rows1, rows2]
        si = [si0, si1, si2]
        sg = [sg0, sg1, sg2]
        so = [so0, so1, so2]
        fetches = [None] * nbuf_n
        gathers = [None] * nbuf_n
        stores = [None] * nbuf_n

        for i in range(nbuf_n):
            fetches[i] = pltpu.async_copy(
                idx_hbm.at[pl.ds(base + i * chunk, chunk)], idx_v[i], si[i])

        def compute(buf, off, size):
            assert size % 64 == 0

            def body(k, carry):
                for u in range(4):
                    s = pl.ds(k * 64 + u * 16, 16)
                    g = lax.iota(jnp.int32, 16) + (off + k * 64 + u * 16)
                    buf[s] = jnp.where(g == 0, 1.0, jnp.exp(buf[s] * _LN2))
                return carry

            lax.fori_loop(0, size // 64, body, 0)

        for t in range(_NS):
            @pl.when(tid == t)
            def _(t=t):
                seq = plan[t]
                ins = [None, None]
                outs = [None, None]
                ins[0] = pltpu.async_copy(
                    table_hbm.at[0, pl.ds(seq[0][0], seq[0][1])],
                    rows_v[0].at[pl.ds(0, seq[0][1])], sg[0])
                for ci, (off, size) in enumerate(seq):
                    b, nbuf = ci % 2, (ci + 1) % 2
                    if ci + 1 < len(seq):
                        noff, nsize = seq[ci + 1]
                        if outs[nbuf] is not None:
                            outs[nbuf].wait()
                        ins[nbuf] = pltpu.async_copy(
                            table_hbm.at[0, pl.ds(noff, nsize)],
                            rows_v[nbuf].at[pl.ds(0, nsize)], sg[nbuf])
                    ins[b].wait()
                    compute(rows_v[b], off, size)
                    outs[b] = pltpu.async_copy(
                        rows_v[b].at[pl.ds(0, size)],
                        shared.at[pl.ds(off, size)], so[b])
                for o in outs:
                    if o is not None:
                        o.wait()

        plsc.subcore_barrier()

        def store_chunk(i, b):
            def body(k, carry):
                g = base_blk + i * bpc + k
                jt = g // nit
                it = g - jt * nit
                obase = jt * (_BJ * nb) + it * _BI
                for jj in range(_BJ):
                    pltpu.async_copy(
                        rows_v[b].at[pl.ds(k * blk + jj * _BI, _BI)],
                        out_hbm.at[pl.ds(obase + jj * nb, _BI)], so[b])
                return carry

            lax.fori_loop(0, bpc, body, 0)

        def wait_store(b):
            pltpu.make_async_copy(
                out_hbm.at[pl.ds(0, chunk)], rows_v[b], so[b]).wait()

        for j in range(nbuf_n - 1):
            fetches[j].wait()
            gathers[j] = pltpu.async_copy(
                shared.at[idx_v[j]], rows_v[j], sg[j])

        for i in range(n_chunks):
            b = i % nbuf_n
            g = i + nbuf_n - 1
            if g < n_chunks:
                gb = g % nbuf_n
                if stores[gb] is not None:
                    wait_store(gb)
                fetches[gb].wait()
                gathers[gb] = pltpu.async_copy(
                    shared.at[idx_v[gb]], rows_v[gb], sg[gb])
            gathers[b].wait()
            store_chunk(i, b)
            stores[b] = True
            if i + nbuf_n < n_chunks:
                fetches[b] = pltpu.async_copy(
                    idx_hbm.at[pl.ds(base + (i + nbuf_n) * chunk, chunk)],
                    idx_v[b], si[b])
        for j in range(nbuf_n):
            if stores[j] is not None:
                wait_store(j)

    return gather_kernel


def kernel(x, table):
    b, l = x.shape
    n = b * l
    v = table.shape[0]
    xb = x.reshape(b // _BI, _BI, l // _BJ, _BJ)
    xb = xb.transpose(2, 0, 3, 1).reshape(-1)
    out = _make_gather(b, l, v, 10240)(table.T, xb)
    return out.reshape(l, b, 1).transpose(1, 0, 2)
